# baseline (speedup 1.0000x reference)
import functools

import jax
import jax.numpy as jnp
from jax import lax
from jax.experimental import pallas as pl
from jax.experimental.pallas import tpu as pltpu
from jax.experimental.pallas import tpu_sc as plsc


def _fused_table_body(emb_ref, w_ref, b_ref, out_ref):
    out_ref[...] = (
        jnp.dot(emb_ref[...], w_ref[...], preferred_element_type=jnp.float32)
        + b_ref[...]
    )


def _make_fused_table(vocab, d_out):
    return pl.pallas_call(
        _fused_table_body,
        out_shape=jax.ShapeDtypeStruct((vocab, d_out), jnp.float32),
    )


def _make_gather(n_tokens, d_out, chunk):
    info = plsc.get_sparse_core_info()
    nw = info.num_cores * info.num_subcores
    per_w = n_tokens // nw
    n_chunks = per_w // chunk
    assert per_w % chunk == 0 and n_tokens % nw == 0

    mesh = plsc.VectorSubcoreMesh(core_axis_name="c", subcore_axis_name="s")

    assert n_chunks >= 4 and n_chunks % 2 == 0

    @functools.partial(
        pl.kernel,
        mesh=mesh,
        out_type=jax.ShapeDtypeStruct((n_tokens, d_out), jnp.float32),
        scratch_types=[
            pltpu.VMEM((2, chunk), jnp.int32),
            pltpu.VMEM((2, chunk, d_out), jnp.float32),
            pltpu.SemaphoreType.DMA,
            pltpu.SemaphoreType.DMA,
            pltpu.SemaphoreType.DMA,
            pltpu.SemaphoreType.DMA,
        ],
    )
    def gather(table_hbm, idx_hbm, out_hbm, idx_v, rows_v, g0, g1, o0, o1):
        wid = lax.axis_index("s") * info.num_cores + lax.axis_index("c")
        base = wid * per_w
        gsem = [g0, g1]
        osem = [o0, o1]

        def idx_load(j, b):
            pltpu.sync_copy(idx_hbm.at[pl.ds(base + j * chunk, chunk)],
                            idx_v.at[b])

        def gather_copy(b, sem):
            return pltpu.make_async_copy(table_hbm.at[idx_v.at[b]],
                                         rows_v.at[b], sem)

        def out_copy(j, b, sem):
            return pltpu.make_async_copy(
                rows_v.at[b], out_hbm.at[pl.ds(base + j * chunk, chunk)], sem)

        for b in range(2):
            idx_load(b, b)
            gather_copy(b, gsem[b]).start()

        def outer(g, carry):
            for b in range(2):
                j = 2 * g + b
                gather_copy(b, gsem[b]).wait()
                out_copy(j, b, osem[b]).start()
                idx_load(j + 2, b)
                out_copy(j, b, osem[b]).wait()
                gather_copy(b, gsem[b]).start()
            return carry

        lax.fori_loop(0, (n_chunks - 2) // 2, outer, 0)

        for b in range(2):
            j = n_chunks - 2 + b
            gather_copy(b, gsem[b]).wait()
            out_copy(j, b, osem[b]).start()
        for b in range(2):
            j = n_chunks - 2 + b
            out_copy(j, b, osem[b]).wait()

    return gather


def kernel(input_ids, emb_table, W, b):
    batch, seqlen = input_ids.shape
    vocab, d_in = emb_table.shape
    d_out = W.shape[1]
    n_tokens = batch * seqlen

    fused = _make_fused_table(vocab, d_out)(emb_table, W, b.reshape(1, d_out))
    ids = input_ids.reshape(n_tokens).astype(jnp.int32)
    out = _make_gather(n_tokens, d_out, 128)(fused, ids)
    return out.reshape(batch, seqlen, d_out)


# speedup vs baseline: 1.2915x; 1.2915x over previous
import functools

import jax
import jax.numpy as jnp
from jax import lax
from jax.experimental import pallas as pl
from jax.experimental.pallas import tpu as pltpu
from jax.experimental.pallas import tpu_sc as plsc


def _fused_table_body(emb_ref, w_ref, b_ref, out_ref):
    out_ref[0] = (
        jnp.dot(emb_ref[...], w_ref[...], preferred_element_type=jnp.float32)
        + b_ref[...]
    )


def _make_fused_table(vocab, d_out, copies):
    return pl.pallas_call(
        _fused_table_body,
        grid=(copies,),
        in_specs=[
            pl.BlockSpec((vocab, 16), lambda i: (0, 0)),
            pl.BlockSpec((16, d_out), lambda i: (0, 0)),
            pl.BlockSpec((1, d_out), lambda i: (0, 0)),
        ],
        out_specs=pl.BlockSpec((1, vocab, d_out), lambda i: (i, 0, 0)),
        out_shape=jax.ShapeDtypeStruct((copies, vocab, d_out), jnp.float32),
    )


def _make_gather(n_tokens, d_out, chunk, vocab):
    info = plsc.get_sparse_core_info()
    nw = info.num_cores * info.num_subcores
    per_w = n_tokens // nw
    n_chunks = per_w // chunk
    assert per_w % chunk == 0 and n_tokens % nw == 0

    mesh = plsc.VectorSubcoreMesh(core_axis_name="c", subcore_axis_name="s")

    assert n_chunks >= 4 and n_chunks % 2 == 0

    @functools.partial(
        pl.kernel,
        mesh=mesh,
        out_type=jax.ShapeDtypeStruct((n_tokens, d_out), jnp.float32),
        scratch_types=[
            pltpu.VMEM((2, chunk), jnp.int32),
            pltpu.VMEM((2, chunk, d_out), jnp.float32),
            pltpu.SemaphoreType.DMA,
            pltpu.SemaphoreType.DMA,
            pltpu.SemaphoreType.DMA,
            pltpu.SemaphoreType.DMA,
        ],
    )
    def gather(table_hbm, idx_hbm, out_hbm, idx_v, rows_v, g0, g1, o0, o1):
        wid = lax.axis_index("s") * info.num_cores + lax.axis_index("c")
        base = wid * per_w
        gsem = [g0, g1]
        osem = [o0, o1]
        row_off = wid * vocab

        def idx_load(j, b):
            pltpu.sync_copy(idx_hbm.at[pl.ds(base + j * chunk, chunk)],
                            idx_v.at[b])
            for k in range(chunk // 16):
                sl = pl.ds(k * 16, 16)
                idx_v[b, sl] = idx_v[b, sl] + row_off

        def gather_copy(b, sem):
            return pltpu.make_async_copy(table_hbm.at[idx_v.at[b]],
                                         rows_v.at[b], sem)

        def out_copy(j, b, sem):
            return pltpu.make_async_copy(
                rows_v.at[b], out_hbm.at[pl.ds(base + j * chunk, chunk)], sem)

        for b in range(2):
            idx_load(b, b)
            gather_copy(b, gsem[b]).start()

        def outer(g, carry):
            for b in range(2):
                j = 2 * g + b
                gather_copy(b, gsem[b]).wait()
                out_copy(j, b, osem[b]).start()
                idx_load(j + 2, b)
                out_copy(j, b, osem[b]).wait()
                gather_copy(b, gsem[b]).start()
            return carry

        lax.fori_loop(0, (n_chunks - 2) // 2, outer, 0)

        for b in range(2):
            j = n_chunks - 2 + b
            gather_copy(b, gsem[b]).wait()
            out_copy(j, b, osem[b]).start()
        for b in range(2):
            j = n_chunks - 2 + b
            out_copy(j, b, osem[b]).wait()

    return gather


def kernel(input_ids, emb_table, W, b):
    batch, seqlen = input_ids.shape
    vocab, d_in = emb_table.shape
    d_out = W.shape[1]
    n_tokens = batch * seqlen

    nw_copies = 32
    fused = _make_fused_table(vocab, d_out, nw_copies)(
        emb_table, W, b.reshape(1, d_out)
    )
    ids = input_ids.reshape(n_tokens).astype(jnp.int32)
    out = _make_gather(n_tokens, d_out, 128, vocab)(
        fused.reshape(nw_copies * vocab, d_out), ids
    )
    return out.reshape(batch, seqlen, d_out)


# speedup vs baseline: 2.0849x; 1.6143x over previous
import functools

import jax
import jax.numpy as jnp
from jax import lax
from jax.experimental import pallas as pl
from jax.experimental.pallas import tpu as pltpu
from jax.experimental.pallas import tpu_sc as plsc


def _fused_table_body(emb_ref, w_ref, b_ref, out_ref):
    out_ref[0] = (
        jnp.dot(emb_ref[...], w_ref[...], preferred_element_type=jnp.float32)
        + b_ref[...]
    )


def _make_fused_table(vocab, d_out, copies):
    return pl.pallas_call(
        _fused_table_body,
        grid=(copies,),
        in_specs=[
            pl.BlockSpec((vocab, 16), lambda i: (0, 0)),
            pl.BlockSpec((16, d_out), lambda i: (0, 0)),
            pl.BlockSpec((1, d_out), lambda i: (0, 0)),
        ],
        out_specs=pl.BlockSpec((1, vocab, d_out), lambda i: (i, 0, 0)),
        out_shape=jax.ShapeDtypeStruct((copies, vocab, d_out), jnp.float32),
    )


def _make_gather(batch, seqlen, d_out, chunk_rows):
    info = plsc.get_sparse_core_info()
    nw = info.num_cores * info.num_subcores
    rows_per_w = batch // nw
    n_chunks = rows_per_w // chunk_rows
    assert batch % nw == 0 and rows_per_w % chunk_rows == 0
    assert seqlen <= 128 and n_chunks >= 4 and n_chunks % 2 == 0

    mesh = plsc.VectorSubcoreMesh(core_axis_name="c", subcore_axis_name="s")

    @functools.partial(
        pl.kernel,
        mesh=mesh,
        out_type=jax.ShapeDtypeStruct((batch, seqlen, d_out), jnp.float32),
        scratch_types=[
            pltpu.VMEM((2, chunk_rows, seqlen), jnp.int32),
            pltpu.VMEM((2, chunk_rows, seqlen, d_out), jnp.float32),
            pltpu.SemaphoreType.DMA,
            pltpu.SemaphoreType.DMA,
            pltpu.SemaphoreType.DMA,
            pltpu.SemaphoreType.DMA,
        ],
    )
    def gather(table_hbm, idx_hbm, out_hbm, idx_v, rows_v, g0, g1, o0, o1):
        wid = lax.axis_index("s") * info.num_cores + lax.axis_index("c")
        row_base = wid * rows_per_w
        gsem = [g0, g1]
        osem = [o0, o1]

        def idx_load(j, b):
            pltpu.sync_copy(
                idx_hbm.at[pl.ds(row_base + j * chunk_rows, chunk_rows)],
                idx_v.at[b])

        def gather_copies(b, sem):
            return [
                pltpu.make_async_copy(
                    table_hbm.at[idx_v.at[b].at[r]],
                    rows_v.at[b].at[r],
                    sem)
                for r in range(chunk_rows)
            ]

        def out_copies(j, b, sem):
            return [
                pltpu.make_async_copy(
                    rows_v.at[b].at[r],
                    out_hbm.at[row_base + j * chunk_rows + r],
                    sem)
                for r in range(chunk_rows)
            ]

        def start(copies):
            for c in copies:
                c.start()

        def wait(copies):
            for c in copies:
                c.wait()

        for b in range(2):
            idx_load(b, b)
            start(gather_copies(b, gsem[b]))

        def outer(g, carry):
            for b in range(2):
                j = 2 * g + b
                wait(gather_copies(b, gsem[b]))
                start(out_copies(j, b, osem[b]))
                idx_load(j + 2, b)
                wait(out_copies(j, b, osem[b]))
                start(gather_copies(b, gsem[b]))
            return carry

        lax.fori_loop(0, (n_chunks - 2) // 2, outer, 0)

        for b in range(2):
            j = n_chunks - 2 + b
            wait(gather_copies(b, gsem[b]))
            start(out_copies(j, b, osem[b]))
        for b in range(2):
            j = n_chunks - 2 + b
            wait(out_copies(j, b, osem[b]))

    return gather


def kernel(input_ids, emb_table, W, b):
    batch, seqlen = input_ids.shape
    vocab, d_in = emb_table.shape
    d_out = W.shape[1]
    n_tokens = batch * seqlen

    info = plsc.get_sparse_core_info()
    nw = info.num_cores * info.num_subcores
    rows_per_w = batch // nw

    fused = _make_fused_table(vocab, d_out, nw)(
        emb_table, W, b.reshape(1, d_out)
    )
    ids = input_ids.astype(jnp.int32) + (
        (jnp.arange(batch, dtype=jnp.int32) // rows_per_w) * vocab
    )[:, None]
    return _make_gather(batch, seqlen, d_out, 4)(
        fused.reshape(nw * vocab, d_out), ids
    )


# speedup vs baseline: 3.6825x; 1.7662x over previous
import functools

import jax
import jax.numpy as jnp
from jax import lax
from jax.experimental import pallas as pl
from jax.experimental.pallas import tpu as pltpu
from jax.experimental.pallas import tpu_sc as plsc


def _fused_table_body(emb_ref, w_ref, b_ref, out_ref):
    out_ref[0] = (
        jnp.dot(emb_ref[...], w_ref[...], preferred_element_type=jnp.float32)
        + b_ref[...]
    )


def _make_fused_table(vocab, d_out, copies):
    return pl.pallas_call(
        _fused_table_body,
        grid=(copies,),
        in_specs=[
            pl.BlockSpec((vocab, 16), lambda i: (0, 0)),
            pl.BlockSpec((16, d_out), lambda i: (0, 0)),
            pl.BlockSpec((1, d_out), lambda i: (0, 0)),
        ],
        out_specs=pl.BlockSpec((1, vocab, d_out), lambda i: (i, 0, 0)),
        out_shape=jax.ShapeDtypeStruct((copies, vocab, d_out), jnp.float32),
    )


def _make_gather(batch, seqlen, d_out, chunk_rows):
    info = plsc.get_sparse_core_info()
    nw = info.num_cores * info.num_subcores
    rows_per_w = batch // nw
    n_chunks = rows_per_w // chunk_rows
    assert batch % nw == 0 and rows_per_w % chunk_rows == 0
    assert seqlen <= 128 and n_chunks >= 4 and n_chunks % 2 == 0

    mesh = plsc.VectorSubcoreMesh(core_axis_name="c", subcore_axis_name="s")

    @functools.partial(
        pl.kernel,
        mesh=mesh,
        out_type=jax.ShapeDtypeStruct((seqlen, batch, d_out), jnp.float32),
        scratch_types=[
            pltpu.VMEM((2, chunk_rows, seqlen), jnp.int32),
            pltpu.VMEM((2, chunk_rows, seqlen, d_out), jnp.float32),
            pltpu.SemaphoreType.DMA,
            pltpu.SemaphoreType.DMA,
            pltpu.SemaphoreType.DMA,
            pltpu.SemaphoreType.DMA,
        ],
    )
    def gather(table_hbm, idx_hbm, out_hbm, idx_v, rows_v, g0, g1, o0, o1):
        wid = lax.axis_index("s") * info.num_cores + lax.axis_index("c")
        row_base = wid * rows_per_w
        gsem = [g0, g1]
        osem = [o0, o1]

        def idx_load(j, b):
            pltpu.sync_copy(
                idx_hbm.at[pl.ds(row_base + j * chunk_rows, chunk_rows)],
                idx_v.at[b])

        def gather_copies(b, sem):
            return [
                pltpu.make_async_copy(
                    table_hbm.at[idx_v.at[b].at[r]],
                    rows_v.at[b].at[r],
                    sem)
                for r in range(chunk_rows)
            ]

        def out_copies(j, b, sem):
            return [
                pltpu.make_async_copy(
                    rows_v.at[b].at[r],
                    out_hbm.at[:, row_base + j * chunk_rows + r, :],
                    sem)
                for r in range(chunk_rows)
            ]

        def start(copies):
            for c in copies:
                c.start()

        def wait(copies):
            for c in copies:
                c.wait()

        for b in range(2):
            idx_load(b, b)
            start(gather_copies(b, gsem[b]))

        def outer(g, carry):
            for b in range(2):
                j = 2 * g + b
                wait(gather_copies(b, gsem[b]))
                start(out_copies(j, b, osem[b]))
                idx_load(j + 2, b)
                wait(out_copies(j, b, osem[b]))
                start(gather_copies(b, gsem[b]))
            return carry

        lax.fori_loop(0, (n_chunks - 2) // 2, outer, 0)

        for b in range(2):
            j = n_chunks - 2 + b
            wait(gather_copies(b, gsem[b]))
            start(out_copies(j, b, osem[b]))
        for b in range(2):
            j = n_chunks - 2 + b
            wait(out_copies(j, b, osem[b]))

    return gather


def kernel(input_ids, emb_table, W, b):
    batch, seqlen = input_ids.shape
    vocab, d_in = emb_table.shape
    d_out = W.shape[1]
    n_tokens = batch * seqlen

    info = plsc.get_sparse_core_info()
    nw = info.num_cores * info.num_subcores
    rows_per_w = batch // nw

    fused = _make_fused_table(vocab, d_out, nw)(
        emb_table, W, b.reshape(1, d_out)
    )
    ids = input_ids.astype(jnp.int32) + (
        (jnp.arange(batch, dtype=jnp.int32) // rows_per_w) * vocab
    )[:, None]
    out = _make_gather(batch, seqlen, d_out, 4)(
        fused.reshape(nw * vocab, d_out), ids
    )
    return out.transpose(1, 0, 2)


# speedup vs baseline: 3.8259x; 1.0389x over previous
import functools

import jax
import jax.numpy as jnp
from jax import lax
from jax.experimental import pallas as pl
from jax.experimental.pallas import tpu as pltpu
from jax.experimental.pallas import tpu_sc as plsc


def _fused_table_body(emb_ref, w_ref, b_ref, out_ref):
    out_ref[0] = (
        jnp.dot(emb_ref[...], w_ref[...], preferred_element_type=jnp.float32)
        + b_ref[...]
    )


def _make_fused_table(vocab, d_out, copies):
    return pl.pallas_call(
        _fused_table_body,
        grid=(copies,),
        in_specs=[
            pl.BlockSpec((vocab, 16), lambda i: (0, 0)),
            pl.BlockSpec((16, d_out), lambda i: (0, 0)),
            pl.BlockSpec((1, d_out), lambda i: (0, 0)),
        ],
        out_specs=pl.BlockSpec((1, vocab, d_out), lambda i: (i, 0, 0)),
        out_shape=jax.ShapeDtypeStruct((copies, vocab, d_out), jnp.float32),
    )


def _make_gather(n_tokens, d_out, chunk):
    info = plsc.get_sparse_core_info()
    nw = info.num_cores * info.num_subcores
    per_w = n_tokens // nw
    n_chunks = per_w // chunk
    assert n_tokens % nw == 0 and per_w % chunk == 0
    assert chunk <= 128 and per_w % 8 == 0 and (chunk * nw) % 8 == 0
    assert n_chunks >= 4 and n_chunks % 2 == 0

    mesh = plsc.VectorSubcoreMesh(core_axis_name="c", subcore_axis_name="s")

    @functools.partial(
        pl.kernel,
        mesh=mesh,
        out_type=jax.ShapeDtypeStruct((n_tokens, d_out), jnp.float32),
        scratch_types=[
            pltpu.VMEM((2, chunk), jnp.int32),
            pltpu.VMEM((2, chunk, d_out), jnp.float32),
            pltpu.SemaphoreType.DMA,
            pltpu.SemaphoreType.DMA,
            pltpu.SemaphoreType.DMA,
            pltpu.SemaphoreType.DMA,
        ],
    )
    def gather(table_hbm, idx_hbm, out_hbm, idx_v, rows_v, g0, g1, o0, o1):
        wid = lax.axis_index("s") * info.num_cores + lax.axis_index("c")
        base = wid * per_w
        gsem = [g0, g1]
        osem = [o0, o1]

        def idx_load(j, b):
            pltpu.sync_copy(idx_hbm.at[pl.ds(base + j * chunk, chunk)],
                            idx_v.at[b])

        def gather_copy(b, sem):
            return pltpu.make_async_copy(table_hbm.at[idx_v.at[b]],
                                         rows_v.at[b], sem)

        def out_copy(j, b, sem):
            return pltpu.make_async_copy(
                rows_v.at[b], out_hbm.at[pl.ds(base + j * chunk, chunk)], sem)

        for b in range(2):
            idx_load(b, b)
            gather_copy(b, gsem[b]).start()

        def outer(g, carry):
            for b in range(2):
                j = 2 * g + b
                gather_copy(b, gsem[b]).wait()
                out_copy(j, b, osem[b]).start()
                idx_load(j + 2, b)
                out_copy(j, b, osem[b]).wait()
                gather_copy(b, gsem[b]).start()
            return carry

        lax.fori_loop(0, (n_chunks - 2) // 2, outer, 0)

        for b in range(2):
            j = n_chunks - 2 + b
            gather_copy(b, gsem[b]).wait()
            out_copy(j, b, osem[b]).start()
        for b in range(2):
            j = n_chunks - 2 + b
            out_copy(j, b, osem[b]).wait()

    return gather


def kernel(input_ids, emb_table, W, b):
    batch, seqlen = input_ids.shape
    vocab, d_in = emb_table.shape
    d_out = W.shape[1]
    n_tokens = batch * seqlen

    info = plsc.get_sparse_core_info()
    nw = info.num_cores * info.num_subcores
    per_w = n_tokens // nw

    fused = _make_fused_table(vocab, d_out, nw)(
        emb_table, W, b.reshape(1, d_out)
    )
    ids = input_ids.astype(jnp.int32).T.reshape(n_tokens)
    ids = ids + (jnp.arange(n_tokens, dtype=jnp.int32) // per_w) * vocab
    out = _make_gather(n_tokens, d_out, 128)(
        fused.reshape(nw * vocab, d_out), ids
    )
    return out.reshape(seqlen, batch, d_out).transpose(1, 0, 2)
